# baseline (speedup 1.0000x reference)
import functools

import jax
import jax.numpy as jnp
from jax import lax
from jax.experimental import pallas as pl
from jax.experimental.pallas import tpu as pltpu
from jax.experimental.pallas import tpu_sc as plsc

NPIX = 196608
H, W, NN = 721, 1440, 4
M = H * W
B = 16
NW = 32
NT = 8112
MP = NT * 128
TPI = 4
CW = TPI * 128
NCHUNK = NT // TPI
ITERS = -(-NCHUNK // NW)
TAIL = M - (NCHUNK - 1) * CW


def _sc_body(xt_hbm, idx_hbm, w_hbm, out_hbm, idx_v, w_v, g_v, out_v,
             sem_g, sem_iw, sem_out):
    wid = lax.axis_index("s") * 2 + lax.axis_index("c")
    lane = lax.iota(jnp.int32, 16)

    def fire_stage(buf, c):
        pltpu.async_copy(idx_hbm.at[pl.ds(c * TPI, TPI)], idx_v.at[buf], sem_iw)
        pltpu.async_copy(w_hbm.at[pl.ds(c * TPI, TPI)], w_v.at[buf], sem_iw)

    def drain_stage(buf):
        pltpu.make_async_copy(idx_hbm.at[pl.ds(0, TPI)], idx_v.at[buf], sem_iw).wait()
        pltpu.make_async_copy(w_hbm.at[pl.ds(0, TPI)], w_v.at[buf], sem_iw).wait()

    def fire_gathers(buf):
        for t in range(TPI):
            for n in range(NN):
                pltpu.async_copy(
                    xt_hbm.at[idx_v.at[buf, t, n]], g_v.at[buf, t, n], sem_g
                )

    def drain_gathers(buf):
        for t in range(TPI):
            for n in range(NN):
                pltpu.make_async_copy(
                    xt_hbm.at[pl.ds(0, 128)], g_v.at[buf, t, n], sem_g
                ).wait()

    def compute(buf):
        @plsc.parallel_loop(0, TPI * 8, unroll=2)
        def grp_body(g):
            t = g >> 3
            jbase = (g & 7) * 16
            wv0 = w_v[buf, t, 0, pl.ds(jbase, 16)]
            wv1 = w_v[buf, t, 1, pl.ds(jbase, 16)]
            wv2 = w_v[buf, t, 2, pl.ds(jbase, 16)]
            wv3 = w_v[buf, t, 3, pl.ds(jbase, 16)]
            colbase = (t << 7) + jbase
            accs = []
            for jj in range(16):
                j = jbase + jj
                a0 = g_v[buf, t, 0, j, :] * jnp.broadcast_to(wv0[jj], (16,))
                a1 = g_v[buf, t, 1, j, :] * jnp.broadcast_to(wv1[jj], (16,))
                a0 += g_v[buf, t, 2, j, :] * jnp.broadcast_to(wv2[jj], (16,))
                a1 += g_v[buf, t, 3, j, :] * jnp.broadcast_to(wv3[jj], (16,))
                accs.append(a0 + a1)
            for jj in range(16):
                plsc.store_scatter(
                    out_v.at[buf],
                    [lane, jnp.full((16,), colbase + jj, jnp.int32)],
                    accs[jj],
                )

    pltpu.sync_copy(idx_hbm.at[pl.ds(wid * TPI, TPI)], idx_v.at[0])
    pltpu.sync_copy(w_hbm.at[pl.ds(wid * TPI, TPI)], w_v.at[0])
    fire_gathers(0)

    def chunk_body(it, carry):
        cur = lax.rem(it, 2)
        nxt = 1 - cur
        c = it * NW + wid
        cn = c + NW
        cp = c - 2 * NW

        @pl.when(cn < NCHUNK)
        def _prefetch():
            fire_stage(nxt, cn)

        @pl.when((it >= 2) & (cp < NCHUNK - 1))
        def _drain_out_full():
            pltpu.make_async_copy(
                out_hbm.at[:, pl.ds(0, CW)], out_v.at[cur], sem_out
            ).wait()

        @pl.when((it >= 2) & (cp == NCHUNK - 1))
        def _drain_out_tail():
            pltpu.make_async_copy(
                out_hbm.at[:, pl.ds(0, TAIL)],
                out_v.at[cur, :, pl.ds(0, TAIL)],
                sem_out,
            ).wait()

        @pl.when(c < NCHUNK)
        def _work():
            drain_gathers(cur)
            compute(cur)

            @pl.when(c < NCHUNK - 1)
            def _out_full():
                pltpu.async_copy(
                    out_v.at[cur], out_hbm.at[:, pl.ds(c * CW, CW)], sem_out
                )

            @pl.when(c == NCHUNK - 1)
            def _out_tail():
                pltpu.async_copy(
                    out_v.at[cur, :, pl.ds(0, TAIL)],
                    out_hbm.at[:, pl.ds(c * CW, TAIL)],
                    sem_out,
                )

        @pl.when(cn < NCHUNK)
        def _next_gathers():
            drain_stage(nxt)
            fire_gathers(nxt)

        return carry

    lax.fori_loop(0, ITERS + 2, chunk_body, 0)


@jax.jit
def _run(xt, idx_t, w_t):
    mesh = plsc.VectorSubcoreMesh(core_axis_name="c", subcore_axis_name="s")
    return pl.kernel(
        _sc_body,
        out_type=jax.ShapeDtypeStruct((B, M), jnp.float32),
        mesh=mesh,
        compiler_params=pltpu.CompilerParams(
            use_tc_tiling_on_sc=False, needs_layout_passes=False
        ),
        scratch_types=[
            pltpu.VMEM((2, TPI, NN, 128), jnp.int32),
            pltpu.VMEM((2, TPI, NN, 128), jnp.float32),
            pltpu.VMEM((2, TPI, NN, 128, B), jnp.float32),
            pltpu.VMEM((2, B, CW), jnp.float32),
            pltpu.SemaphoreType.DMA,
            pltpu.SemaphoreType.DMA,
            pltpu.SemaphoreType.DMA,
        ],
    )(xt, idx_t, w_t)


def kernel(x, index, weight):
    batch = x.shape[:-1]
    fone = weight[0, 0] * 0.0 + 1.0
    xt = x.reshape(-1, NPIX).T * fone
    idx_p = jnp.concatenate([index, jnp.zeros((MP - M, NN), jnp.int32)], axis=0)
    w_p = jnp.concatenate([weight, jnp.zeros((MP - M, NN), jnp.float32)], axis=0)
    idx_t = idx_p.reshape(NT, 128, NN).transpose(0, 2, 1)
    w_t = w_p.reshape(NT, 128, NN).transpose(0, 2, 1) * fone
    out = _run(xt, idx_t, w_t)
    out_wh = lax.optimization_barrier(out.reshape(B, H, W).transpose(0, 2, 1))
    return out_wh.transpose(0, 2, 1).reshape(batch + (H, W))


# speedup vs baseline: 1.2281x; 1.2281x over previous
import functools

import jax
import jax.numpy as jnp
from jax import lax
from jax.experimental import pallas as pl
from jax.experimental.pallas import tpu as pltpu
from jax.experimental.pallas import tpu_sc as plsc

NPIX = 196608
H, W, NN = 721, 1440, 4
M = H * W
B = 16
NW = 32
NT = 8112
MP = NT * 128
TPI = 4
CW = TPI * 128
NCHUNK = NT // TPI
ITERS = -(-NCHUNK // NW)
TAIL = M - (NCHUNK - 1) * CW


def _sc_body(xt_hbm, idx_hbm, w_hbm, out_hbm, idx_v, w_v, g_v, out_v,
             sem_g, sem_iw, sem_out):
    wid = lax.axis_index("s") * 2 + lax.axis_index("c")
    lane = lax.iota(jnp.int32, 16)

    def fire_stage(buf, c):
        pltpu.async_copy(idx_hbm.at[pl.ds(c * TPI, TPI)], idx_v.at[buf], sem_iw)
        pltpu.async_copy(w_hbm.at[pl.ds(c * TPI, TPI)], w_v.at[buf], sem_iw)

    def drain_stage(buf):
        pltpu.make_async_copy(idx_hbm.at[pl.ds(0, TPI)], idx_v.at[buf], sem_iw).wait()
        pltpu.make_async_copy(w_hbm.at[pl.ds(0, TPI)], w_v.at[buf], sem_iw).wait()

    def fire_gathers(buf):
        for t in range(TPI):
            for n in range(NN):
                pltpu.async_copy(
                    xt_hbm.at[idx_v.at[buf, t, n]], g_v.at[buf, t, n], sem_g
                )

    def drain_gathers(buf):
        for t in range(TPI):
            for n in range(NN):
                pltpu.make_async_copy(
                    xt_hbm.at[pl.ds(0, 128)], g_v.at[buf, t, n], sem_g
                ).wait()

    def compute(buf):
        @plsc.parallel_loop(0, TPI * 8, unroll=2)
        def grp_body(g):
            t = g >> 3
            jbase = (g & 7) * 16
            wv0 = w_v[buf, t, 0, pl.ds(jbase, 16)]
            wv1 = w_v[buf, t, 1, pl.ds(jbase, 16)]
            wv2 = w_v[buf, t, 2, pl.ds(jbase, 16)]
            wv3 = w_v[buf, t, 3, pl.ds(jbase, 16)]
            colbase = (t << 7) + jbase
            for q in range(4):
                accs = []
                for jj in range(q * 4, q * 4 + 4):
                    j = jbase + jj
                    a0 = g_v[buf, t, 0, j, :] * jnp.broadcast_to(wv0[jj], (16,))
                    a1 = g_v[buf, t, 1, j, :] * jnp.broadcast_to(wv1[jj], (16,))
                    a0 += g_v[buf, t, 2, j, :] * jnp.broadcast_to(wv2[jj], (16,))
                    a1 += g_v[buf, t, 3, j, :] * jnp.broadcast_to(wv3[jj], (16,))
                    accs.append(a0 + a1)
                for i, jj in enumerate(range(q * 4, q * 4 + 4)):
                    plsc.store_scatter(
                        out_v.at[buf],
                        [lane, jnp.full((16,), colbase + jj, jnp.int32)],
                        accs[i],
                    )

    pltpu.sync_copy(idx_hbm.at[pl.ds(wid * TPI, TPI)], idx_v.at[0])
    pltpu.sync_copy(w_hbm.at[pl.ds(wid * TPI, TPI)], w_v.at[0])
    fire_gathers(0)

    def chunk_body(it, carry):
        cur = lax.rem(it, 2)
        nxt = 1 - cur
        c = it * NW + wid
        cn = c + NW
        cp = c - 2 * NW

        @pl.when(cn < NCHUNK)
        def _prefetch():
            fire_stage(nxt, cn)

        @pl.when((it >= 2) & (cp < NCHUNK - 1))
        def _drain_out_full():
            pltpu.make_async_copy(
                out_hbm.at[:, pl.ds(0, CW)], out_v.at[cur], sem_out
            ).wait()

        @pl.when((it >= 2) & (cp == NCHUNK - 1))
        def _drain_out_tail():
            pltpu.make_async_copy(
                out_hbm.at[:, pl.ds(0, TAIL)],
                out_v.at[cur, :, pl.ds(0, TAIL)],
                sem_out,
            ).wait()

        @pl.when(c < NCHUNK)
        def _work():
            drain_gathers(cur)
            compute(cur)

            @pl.when(c < NCHUNK - 1)
            def _out_full():
                pltpu.async_copy(
                    out_v.at[cur], out_hbm.at[:, pl.ds(c * CW, CW)], sem_out
                )

            @pl.when(c == NCHUNK - 1)
            def _out_tail():
                pltpu.async_copy(
                    out_v.at[cur, :, pl.ds(0, TAIL)],
                    out_hbm.at[:, pl.ds(c * CW, TAIL)],
                    sem_out,
                )

        @pl.when(cn < NCHUNK)
        def _next_gathers():
            drain_stage(nxt)
            fire_gathers(nxt)

        return carry

    lax.fori_loop(0, ITERS + 2, chunk_body, 0)


@jax.jit
def _run(xt, idx_t, w_t):
    mesh = plsc.VectorSubcoreMesh(core_axis_name="c", subcore_axis_name="s")
    return pl.kernel(
        _sc_body,
        out_type=jax.ShapeDtypeStruct((B, M), jnp.float32),
        mesh=mesh,
        compiler_params=pltpu.CompilerParams(
            use_tc_tiling_on_sc=False, needs_layout_passes=False
        ),
        scratch_types=[
            pltpu.VMEM((2, TPI, NN, 128), jnp.int32),
            pltpu.VMEM((2, TPI, NN, 128), jnp.float32),
            pltpu.VMEM((2, TPI, NN, 128, B), jnp.float32),
            pltpu.VMEM((2, B, CW), jnp.float32),
            pltpu.SemaphoreType.DMA,
            pltpu.SemaphoreType.DMA,
            pltpu.SemaphoreType.DMA,
        ],
    )(xt, idx_t, w_t)


def kernel(x, index, weight):
    batch = x.shape[:-1]
    fone = weight[0, 0] * 0.0 + 1.0
    xt = x.reshape(-1, NPIX).T * fone
    idx_p = jnp.concatenate([index, jnp.zeros((MP - M, NN), jnp.int32)], axis=0)
    w_p = jnp.concatenate([weight, jnp.zeros((MP - M, NN), jnp.float32)], axis=0)
    idx_t = idx_p.reshape(NT, 128, NN).transpose(0, 2, 1)
    w_t = w_p.reshape(NT, 128, NN).transpose(0, 2, 1) * fone
    out = _run(xt, idx_t, w_t)
    out_wh = lax.optimization_barrier(out.reshape(B, H, W).transpose(0, 2, 1))
    return out_wh.transpose(0, 2, 1).reshape(batch + (H, W))


# speedup vs baseline: 1.4806x; 1.2057x over previous
import functools

import jax
import jax.numpy as jnp
from jax import lax
from jax.experimental import pallas as pl
from jax.experimental.pallas import tpu as pltpu
from jax.experimental.pallas import tpu_sc as plsc

NPIX = 196608
H, W, NN = 721, 1440, 4
M = H * W
B = 16
NW = 32
NT = 8112
MP = NT * 128
TPI = 4
CW = TPI * 128
NCHUNK = NT // TPI
ITERS = -(-NCHUNK // NW)
TAIL = M - (NCHUNK - 1) * CW


def _sc_body(xt_hbm, idx_hbm, w_hbm, out_hbm, idx_v, w_v, g_v, out_v,
             sem_g, sem_idx, sem_w, sem_out):
    wid = lax.axis_index("s") * 2 + lax.axis_index("c")
    lane = lax.iota(jnp.int32, 16)

    def fire_idx(ib, c):
        pltpu.async_copy(idx_hbm.at[pl.ds(c * TPI, TPI)], idx_v.at[ib], sem_idx)

    def drain_idx(ib):
        pltpu.make_async_copy(idx_hbm.at[pl.ds(0, TPI)], idx_v.at[ib], sem_idx).wait()

    def fire_w(buf, c):
        pltpu.async_copy(w_hbm.at[pl.ds(c * TPI, TPI)], w_v.at[buf], sem_w)

    def drain_w(buf):
        pltpu.make_async_copy(w_hbm.at[pl.ds(0, TPI)], w_v.at[buf], sem_w).wait()

    def fire_gathers(ib, buf):
        for t in range(TPI):
            for n in range(NN):
                pltpu.async_copy(
                    xt_hbm.at[idx_v.at[ib, t, n]], g_v.at[buf, t, n], sem_g
                )

    def drain_gathers(buf):
        for t in range(TPI):
            for n in range(NN):
                pltpu.make_async_copy(
                    xt_hbm.at[pl.ds(0, 128)], g_v.at[buf, t, n], sem_g
                ).wait()

    def compute(buf):
        @plsc.parallel_loop(0, TPI * 8, unroll=2)
        def grp_body(g):
            t = g >> 3
            jbase = (g & 7) * 16
            wv0 = w_v[buf, t, 0, pl.ds(jbase, 16)]
            wv1 = w_v[buf, t, 1, pl.ds(jbase, 16)]
            wv2 = w_v[buf, t, 2, pl.ds(jbase, 16)]
            wv3 = w_v[buf, t, 3, pl.ds(jbase, 16)]
            colbase = (t << 7) + jbase
            for q in range(4):
                accs = []
                for jj in range(q * 4, q * 4 + 4):
                    j = jbase + jj
                    a0 = g_v[buf, t, 0, j, :] * jnp.broadcast_to(wv0[jj], (16,))
                    a1 = g_v[buf, t, 1, j, :] * jnp.broadcast_to(wv1[jj], (16,))
                    a0 += g_v[buf, t, 2, j, :] * jnp.broadcast_to(wv2[jj], (16,))
                    a1 += g_v[buf, t, 3, j, :] * jnp.broadcast_to(wv3[jj], (16,))
                    accs.append(a0 + a1)
                for i, jj in enumerate(range(q * 4, q * 4 + 4)):
                    plsc.store_scatter(
                        out_v.at[buf],
                        [lane, jnp.full((16,), colbase + jj, jnp.int32)],
                        accs[i],
                    )

    pltpu.sync_copy(idx_hbm.at[pl.ds(wid * TPI, TPI)], idx_v.at[0])
    fire_gathers(0, 0)
    fire_idx(1, wid + NW)
    fire_w(0, wid)

    def chunk_body(it, carry):
        cur = lax.rem(it, 2)
        i_nx = lax.rem(it + 1, 3)
        i_nn = lax.rem(it + 2, 3)
        c = it * NW + wid
        cn = c + NW
        cnn = c + 2 * NW
        cp = c - 2 * NW

        @pl.when(cn < NCHUNK)
        def _stage_next_w():
            fire_w(1 - cur, cn)

        @pl.when(cn < NCHUNK)
        def _next_gathers():
            drain_idx(i_nx)
            fire_gathers(i_nx, 1 - cur)

        @pl.when(cnn < NCHUNK)
        def _prefetch_idx():
            fire_idx(i_nn, cnn)

        @pl.when((it >= 2) & (cp < NCHUNK - 1))
        def _drain_out_full():
            pltpu.make_async_copy(
                out_hbm.at[:, pl.ds(0, CW)], out_v.at[cur], sem_out
            ).wait()

        @pl.when((it >= 2) & (cp == NCHUNK - 1))
        def _drain_out_tail():
            pltpu.make_async_copy(
                out_hbm.at[:, pl.ds(0, TAIL)],
                out_v.at[cur, :, pl.ds(0, TAIL)],
                sem_out,
            ).wait()

        @pl.when(c < NCHUNK)
        def _work():
            drain_w(cur)
            drain_gathers(cur)
            compute(cur)

            @pl.when(c < NCHUNK - 1)
            def _out_full():
                pltpu.async_copy(
                    out_v.at[cur], out_hbm.at[:, pl.ds(c * CW, CW)], sem_out
                )

            @pl.when(c == NCHUNK - 1)
            def _out_tail():
                pltpu.async_copy(
                    out_v.at[cur, :, pl.ds(0, TAIL)],
                    out_hbm.at[:, pl.ds(c * CW, TAIL)],
                    sem_out,
                )

        return carry

    lax.fori_loop(0, ITERS + 2, chunk_body, 0)


@jax.jit
def _run(xt, idx_t, w_t):
    mesh = plsc.VectorSubcoreMesh(core_axis_name="c", subcore_axis_name="s")
    return pl.kernel(
        _sc_body,
        out_type=jax.ShapeDtypeStruct((B, M), jnp.float32),
        mesh=mesh,
        compiler_params=pltpu.CompilerParams(
            use_tc_tiling_on_sc=False, needs_layout_passes=False
        ),
        scratch_types=[
            pltpu.VMEM((3, TPI, NN, 128), jnp.int32),
            pltpu.VMEM((2, TPI, NN, 128), jnp.float32),
            pltpu.VMEM((2, TPI, NN, 128, B), jnp.float32),
            pltpu.VMEM((2, B, CW), jnp.float32),
            pltpu.SemaphoreType.DMA,
            pltpu.SemaphoreType.DMA,
            pltpu.SemaphoreType.DMA,
            pltpu.SemaphoreType.DMA,
        ],
    )(xt, idx_t, w_t)


def kernel(x, index, weight):
    batch = x.shape[:-1]
    fone = weight[0, 0] * 0.0 + 1.0
    xt = x.reshape(-1, NPIX).T * fone
    idx_p = jnp.concatenate([index, jnp.zeros((MP - M, NN), jnp.int32)], axis=0)
    w_p = jnp.concatenate([weight, jnp.zeros((MP - M, NN), jnp.float32)], axis=0)
    idx_t = idx_p.reshape(NT, 128, NN).transpose(0, 2, 1)
    w_t = w_p.reshape(NT, 128, NN).transpose(0, 2, 1) * fone
    out = _run(xt, idx_t, w_t)
    out_wh = lax.optimization_barrier(out.reshape(B, H, W).transpose(0, 2, 1))
    return out_wh.transpose(0, 2, 1).reshape(batch + (H, W))
